# baseline (speedup 1.0000x reference)
import jax
import jax.numpy as jnp
from jax.experimental import pallas as pl


_TM = 2048

_DN_T_RHS = (((1,), (1,)), ((), ()))
_DN_T_LHS = (((0,), (1,)), ((), ()))
_DN_COL = (((0,), (0,)), ((), ()))


def _proj_body(emb_ref, smt_ref, w_ref, b_ref, out_ref):
    emb_dim = emb_ref.shape[1]
    n_small = smt_ref.shape[0] - 1
    acc = jax.lax.dot_general(emb_ref[...].astype(jnp.bfloat16),
                              w_ref[:, :emb_dim].astype(jnp.bfloat16),
                              _DN_T_RHS, preferred_element_type=jnp.float32)
    w_small = w_ref[:, emb_dim:].astype(jnp.bfloat16)
    acc += jax.lax.dot_general(smt_ref[:n_small, :], w_small, _DN_T_LHS,
                               preferred_element_type=jnp.float32)
    acc += b_ref[...]
    mcol = jax.lax.dot_general(smt_ref[n_small:, :],
                               jnp.ones((1, 1), jnp.bfloat16), _DN_COL,
                               preferred_element_type=jnp.float32)
    out_ref[...] = acc * mcol


def kernel(embeddings, visibility_scores, bbox_ltwh, keypoints_xyc, feats_masks, W, b):
    B, N = feats_masks.shape
    M = B * N
    emb_dim = embeddings.shape[-1]
    kp_dim = keypoints_xyc.shape[-2] * keypoints_xyc.shape[-1]
    token_dim = W.shape[0]

    emb = embeddings.reshape(M, emb_dim)
    smallT = jnp.concatenate(
        [visibility_scores.reshape(M, 1),
         bbox_ltwh.reshape(M, 4),
         keypoints_xyc.reshape(M, kp_dim),
         feats_masks.reshape(M, 1).astype(jnp.float32)],
        axis=1).astype(jnp.bfloat16).T
    b2 = b.reshape(1, token_dim)

    grid = (M // _TM,)
    out = pl.pallas_call(
        _proj_body,
        grid=grid,
        in_specs=[
            pl.BlockSpec((_TM, emb_dim), lambda i: (i, 0)),
            pl.BlockSpec((kp_dim + 6, _TM), lambda i: (0, i)),
            pl.BlockSpec(W.shape, lambda i: (0, 0)),
            pl.BlockSpec(b2.shape, lambda i: (0, 0)),
        ],
        out_specs=pl.BlockSpec((_TM, token_dim), lambda i: (i, 0)),
        out_shape=jax.ShapeDtypeStruct((M, token_dim), jnp.float32),
    )(emb, smallT, W, b2)

    return out.reshape(B, N, token_dim)
